# baseline (speedup 1.0000x reference)
import functools

import jax
import jax.numpy as jnp
from jax.experimental import pallas as pl
from jax.experimental.pallas import tpu as pltpu

B, S, D = 1, 2048, 1024
E, K, F = 8, 2, 2048
T = B * S

BS = 256
FB = 512
NF = F // FB
NB = (T * K) // BS + E
PADDED = NB * BS


def _gate_body(x_ref, gw_ref, i1_ref, i2_ref, w1_ref, w2_ref):
    x = x_ref[...]
    gw = gw_ref[...]
    logits = jax.lax.dot_general(
        x, gw, (((1,), (1,)), ((), ())), preferred_element_type=jnp.float32
    )
    iota = jax.lax.broadcasted_iota(jnp.int32, logits.shape, 1)
    m1 = jnp.max(logits, axis=1, keepdims=True)
    i1 = jnp.min(jnp.where(logits == m1, iota, E), axis=1, keepdims=True)
    masked = jnp.where(iota == i1, -jnp.inf, logits)
    m2 = jnp.max(masked, axis=1, keepdims=True)
    i2 = jnp.min(jnp.where(masked == m2, iota, E), axis=1, keepdims=True)
    wa = jax.nn.sigmoid(m1 - m2)
    i1_ref[...] = i1
    i2_ref[...] = i2
    w1_ref[...] = wa
    w2_ref[...] = 1.0 - wa


def _moe_body(be_ref, ids_ref, valid_ref, w1_ref, w3_ref, w2_ref, x_ref,
              sw_ref, out_ref, xs_ref, ys_ref, acc_ref):
    f = pl.program_id(0)
    b = pl.program_id(1)

    @pl.when(jnp.logical_and(f == 0, b == 0))
    def _():
        out_ref[...] = jnp.zeros_like(out_ref)

    base = pl.multiple_of(b * BS, BS)
    is_valid = valid_ref[b] > 0

    @pl.when(jnp.logical_and(f == 0, is_valid))
    def _():
        def gather(i, _):
            t = ids_ref[b * BS + i]
            acc_ref[pl.ds(i, 1), :] = x_ref[pl.ds(t, 1), :]
            return 0
        jax.lax.fori_loop(0, BS, gather, 0)
        xs_ref[pl.ds(base, BS), :] = acc_ref[...].astype(jnp.bfloat16)

    @pl.when(is_valid)
    def _():
        xb = xs_ref[pl.ds(base, BS), :]
        w1b = w1_ref[0].astype(jnp.bfloat16)
        w3b = w3_ref[0].astype(jnp.bfloat16)
        w2b = w2_ref[0].astype(jnp.bfloat16)
        h1 = jax.lax.dot_general(
            xb, w1b, (((1,), (1,)), ((), ())), preferred_element_type=jnp.float32)
        h3 = jax.lax.dot_general(
            xb, w3b, (((1,), (1,)), ((), ())), preferred_element_type=jnp.float32)
        h = (h1 * jax.nn.sigmoid(h1) * h3).astype(jnp.bfloat16)
        y = jax.lax.dot_general(
            h, w2b, (((1,), (1,)), ((), ())), preferred_element_type=jnp.float32)

        @pl.when(f == 0)
        def _():
            ys_ref[pl.ds(base, BS), :] = y.astype(jnp.bfloat16)

        @pl.when(f > 0)
        def _():
            prev = ys_ref[pl.ds(base, BS), :].astype(jnp.float32)
            tot = prev + y

            @pl.when(f < NF - 1)
            def _():
                ys_ref[pl.ds(base, BS), :] = tot.astype(jnp.bfloat16)

            @pl.when(f == NF - 1)
            def _():
                acc_ref[...] = tot * sw_ref[...]

                def scatter(i, _):
                    t = ids_ref[b * BS + i]
                    out_ref[pl.ds(t, 1), :] += acc_ref[pl.ds(i, 1), :]
                    return 0
                jax.lax.fori_loop(0, BS, scatter, 0)


def _run_main(x, w1, w3, w2, sort_ids, slot_w, block_expert, block_valid,
              orig_shape):
    grid_spec = pltpu.PrefetchScalarGridSpec(
        num_scalar_prefetch=3,
        grid=(NF, NB),
        in_specs=[
            pl.BlockSpec((1, FB, D), lambda f, b, be, ids, va: (be[b], f, 0)),
            pl.BlockSpec((1, FB, D), lambda f, b, be, ids, va: (be[b], f, 0)),
            pl.BlockSpec((1, D, FB), lambda f, b, be, ids, va: (be[b], 0, f)),
            pl.BlockSpec((T, D), lambda f, b, be, ids, va: (0, 0)),
            pl.BlockSpec((BS, 1), lambda f, b, be, ids, va: (b, 0)),
        ],
        out_specs=pl.BlockSpec((T, D), lambda f, b, be, ids, va: (0, 0)),
        scratch_shapes=[
            pltpu.VMEM((PADDED, D), jnp.bfloat16),
            pltpu.VMEM((PADDED, D), jnp.bfloat16),
            pltpu.VMEM((BS, D), jnp.float32),
        ],
    )
    y = pl.pallas_call(
        _moe_body,
        grid_spec=grid_spec,
        out_shape=jax.ShapeDtypeStruct((T, D), jnp.float32),
        compiler_params=pltpu.CompilerParams(
            dimension_semantics=("arbitrary", "arbitrary"),
        ),
    )(block_expert, sort_ids, block_valid, w1, w3, w2, x, slot_w[:, None])
    return y.reshape(orig_shape)


@jax.jit
def kernel(hidden_states, gate_w, w1, w3, w2):
    orig_shape = hidden_states.shape
    x = hidden_states.reshape(T, D)

    i1, i2, wa, wb = pl.pallas_call(
        _gate_body,
        out_shape=(
            jax.ShapeDtypeStruct((T, 1), jnp.int32),
            jax.ShapeDtypeStruct((T, 1), jnp.int32),
            jax.ShapeDtypeStruct((T, 1), jnp.float32),
            jax.ShapeDtypeStruct((T, 1), jnp.float32),
        ),
    )(x, gate_w)

    flat_e = jnp.concatenate([i1, i2], axis=1).reshape(-1)
    flat_w = jnp.concatenate([wa, wb], axis=1).reshape(-1)
    oh = (flat_e[:, None] == jnp.arange(E)[None, :]).astype(jnp.int32)
    counts = jnp.sum(oh, axis=0)
    padded = ((counts + BS - 1) // BS) * BS
    offs = jnp.concatenate([jnp.zeros(1, jnp.int32),
                            jnp.cumsum(padded)[:-1].astype(jnp.int32)])
    rank = jnp.cumsum(oh, axis=0) - 1
    my_rank = jnp.take_along_axis(rank, flat_e[:, None], axis=1)[:, 0]
    pos = offs[flat_e] + my_rank
    sort_ids = jnp.zeros(PADDED, jnp.int32).at[pos].set(
        jnp.arange(T * K, dtype=jnp.int32) // K)
    slot_w = jnp.zeros(PADDED, jnp.float32).at[pos].set(flat_w)
    block_starts = jnp.arange(NB, dtype=jnp.int32) * BS
    block_expert = jnp.sum(
        block_starts[:, None] >= offs[None, :], axis=1, dtype=jnp.int32) - 1
    total = offs[E - 1] + padded[E - 1]
    block_valid = (block_starts < total).astype(jnp.int32)
    return _run_main(x, w1, w3, w2, sort_ids, slot_w, block_expert,
                     block_valid, orig_shape)


# speedup vs baseline: 1.0399x; 1.0399x over previous
import functools

import jax
import jax.numpy as jnp
from jax.experimental import pallas as pl
from jax.experimental.pallas import tpu as pltpu

B, S, D = 1, 2048, 1024
E, K, F = 8, 2, 2048
T = B * S

BS = 256
FB = 1024
NF = F // FB
NB = (T * K) // BS + E
PADDED = NB * BS


def _gate_body(x_ref, gw_ref, i1_ref, i2_ref, w1_ref, w2_ref):
    x = x_ref[...]
    gw = gw_ref[...]
    logits = jax.lax.dot_general(
        x, gw, (((1,), (1,)), ((), ())), preferred_element_type=jnp.float32
    )
    iota = jax.lax.broadcasted_iota(jnp.int32, logits.shape, 1)
    m1 = jnp.max(logits, axis=1, keepdims=True)
    i1 = jnp.min(jnp.where(logits == m1, iota, E), axis=1, keepdims=True)
    masked = jnp.where(iota == i1, -jnp.inf, logits)
    m2 = jnp.max(masked, axis=1, keepdims=True)
    i2 = jnp.min(jnp.where(masked == m2, iota, E), axis=1, keepdims=True)
    wa = jax.nn.sigmoid(m1 - m2)
    i1_ref[...] = i1
    i2_ref[...] = i2
    w1_ref[...] = wa
    w2_ref[...] = 1.0 - wa


def _dispatch_body(ids_ref, x_ref, xs_ref, row_ref):
    def gather(i, _):
        t = ids_ref[i]
        row_ref[pl.ds(i % 8, 1), :] = x_ref[pl.ds(t, 1), :]

        @pl.when(i % 8 == 7)
        def _():
            base = pl.multiple_of(i - 7, 8)
            xs_ref[pl.ds(base, 8), :] = row_ref[...].astype(jnp.bfloat16)
        return 0
    jax.lax.fori_loop(0, PADDED, gather, 0)


def _moe_body(be_ref, ids_ref, valid_ref, w1_ref, w3_ref, w2_ref, xs_ref,
              sw_ref, out_ref, ys_ref, acc_ref):
    f = pl.program_id(0)
    b = pl.program_id(1)

    @pl.when(jnp.logical_and(f == 0, b == 0))
    def _():
        out_ref[...] = jnp.zeros_like(out_ref)

    base = pl.multiple_of(b * BS, BS)
    is_valid = valid_ref[b] > 0

    @pl.when(is_valid)
    def _():
        xb = xs_ref[0]
        w1b = w1_ref[0].astype(jnp.bfloat16)
        w3b = w3_ref[0].astype(jnp.bfloat16)
        w2b = w2_ref[0].astype(jnp.bfloat16)
        h1 = jax.lax.dot_general(
            xb, w1b, (((1,), (1,)), ((), ())), preferred_element_type=jnp.float32)
        h3 = jax.lax.dot_general(
            xb, w3b, (((1,), (1,)), ((), ())), preferred_element_type=jnp.float32)
        h = (h1 * jax.nn.sigmoid(h1) * h3).astype(jnp.bfloat16)
        y = jax.lax.dot_general(
            h, w2b, (((1,), (1,)), ((), ())), preferred_element_type=jnp.float32)

        @pl.when(f == 0)
        def _():
            ys_ref[pl.ds(base, BS), :] = y.astype(jnp.bfloat16)

        @pl.when(f > 0)
        def _():
            prev = ys_ref[pl.ds(base, BS), :].astype(jnp.float32)
            tot = prev + y

            @pl.when(f < NF - 1)
            def _():
                ys_ref[pl.ds(base, BS), :] = tot.astype(jnp.bfloat16)

            @pl.when(f == NF - 1)
            def _():
                acc_ref[...] = tot * sw_ref[...]

                def scatter(i, _):
                    t = ids_ref[b * BS + i]
                    out_ref[pl.ds(t, 1), :] += acc_ref[pl.ds(i, 1), :]
                    return 0
                jax.lax.fori_loop(0, BS, scatter, 0)


def _run_main(x, w1, w3, w2, sort_ids, slot_w, block_expert, block_valid,
              orig_shape):
    xs = pl.pallas_call(
        _dispatch_body,
        grid_spec=pltpu.PrefetchScalarGridSpec(
            num_scalar_prefetch=1,
            grid=(1,),
            in_specs=[pl.BlockSpec((T, D), lambda i, ids: (0, 0))],
            out_specs=pl.BlockSpec((PADDED, D), lambda i, ids: (0, 0)),
            scratch_shapes=[pltpu.VMEM((8, D), jnp.float32)],
        ),
        out_shape=jax.ShapeDtypeStruct((PADDED, D), jnp.bfloat16),
    )(sort_ids, x)

    grid_spec = pltpu.PrefetchScalarGridSpec(
        num_scalar_prefetch=3,
        grid=(NF, NB),
        in_specs=[
            pl.BlockSpec((1, FB, D), lambda f, b, be, ids, va: (be[b], f, 0)),
            pl.BlockSpec((1, FB, D), lambda f, b, be, ids, va: (be[b], f, 0)),
            pl.BlockSpec((1, D, FB), lambda f, b, be, ids, va: (be[b], 0, f)),
            pl.BlockSpec((1, BS, D), lambda f, b, be, ids, va: (0, b, 0)),
            pl.BlockSpec((BS, 1), lambda f, b, be, ids, va: (b, 0)),
        ],
        out_specs=pl.BlockSpec((T, D), lambda f, b, be, ids, va: (0, 0)),
        scratch_shapes=[
            pltpu.VMEM((PADDED, D), jnp.bfloat16),
            pltpu.VMEM((BS, D), jnp.float32),
        ],
    )
    y = pl.pallas_call(
        _moe_body,
        grid_spec=grid_spec,
        out_shape=jax.ShapeDtypeStruct((T, D), jnp.float32),
        compiler_params=pltpu.CompilerParams(
            dimension_semantics=("arbitrary", "arbitrary"),
        ),
    )(block_expert, sort_ids, block_valid, w1, w3, w2,
      xs.reshape(NB, BS, D), slot_w[:, None])
    return y.reshape(orig_shape)


@jax.jit
def kernel(hidden_states, gate_w, w1, w3, w2):
    orig_shape = hidden_states.shape
    x = hidden_states.reshape(T, D)

    i1, i2, wa, wb = pl.pallas_call(
        _gate_body,
        out_shape=(
            jax.ShapeDtypeStruct((T, 1), jnp.int32),
            jax.ShapeDtypeStruct((T, 1), jnp.int32),
            jax.ShapeDtypeStruct((T, 1), jnp.float32),
            jax.ShapeDtypeStruct((T, 1), jnp.float32),
        ),
    )(x, gate_w)

    DIAG = True
    if DIAG:
        sort_ids = (jnp.arange(PADDED, dtype=jnp.int32) % T) + i1[0, 0] * 0
        slot_w = jnp.ones(PADDED, jnp.float32) * 0.5
        block_expert = jnp.arange(NB, dtype=jnp.int32) // 3
        block_valid = jnp.ones(NB, jnp.int32)
        return _run_main(x, w1, w3, w2, sort_ids, slot_w, block_expert,
                         block_valid, orig_shape)
    flat_e = jnp.concatenate([i1, i2], axis=1).reshape(-1)
    flat_w = jnp.concatenate([wa, wb], axis=1).reshape(-1)
    oh = (flat_e[:, None] == jnp.arange(E)[None, :]).astype(jnp.int32)
    counts = jnp.sum(oh, axis=0)
    padded = ((counts + BS - 1) // BS) * BS
    offs = jnp.concatenate([jnp.zeros(1, jnp.int32),
                            jnp.cumsum(padded)[:-1].astype(jnp.int32)])
    rank = jnp.cumsum(oh, axis=0) - 1
    my_rank = jnp.take_along_axis(rank, flat_e[:, None], axis=1)[:, 0]
    pos = offs[flat_e] + my_rank
    sort_ids = jnp.zeros(PADDED, jnp.int32).at[pos].set(
        jnp.arange(T * K, dtype=jnp.int32) // K)
    slot_w = jnp.zeros(PADDED, jnp.float32).at[pos].set(flat_w)
    block_starts = jnp.arange(NB, dtype=jnp.int32) * BS
    block_expert = jnp.sum(
        block_starts[:, None] >= offs[None, :], axis=1, dtype=jnp.int32) - 1
    total = offs[E - 1] + padded[E - 1]
    block_valid = (block_starts < total).astype(jnp.int32)
    return _run_main(x, w1, w3, w2, sort_ids, slot_w, block_expert,
                     block_valid, orig_shape)


# speedup vs baseline: 1.1541x; 1.1098x over previous
import functools

import jax
import jax.numpy as jnp
from jax.experimental import pallas as pl
from jax.experimental.pallas import tpu as pltpu

B, S, D = 1, 2048, 1024
E, K, F = 8, 2, 2048
T = B * S

BS = 256
FB = 1024
NF = F // FB
NB = (T * K) // BS + E
PADDED = NB * BS
CH = 512
NCH = (T * K) // CH


def _gate_body(x_ref, gw_ref, ids_ref, sw_ref, be_ref, valid_ref):
    x = x_ref[...]
    gw = gw_ref[...]
    logits = jax.lax.dot_general(
        x, gw, (((1,), (1,)), ((), ())), preferred_element_type=jnp.float32
    )
    iota_e = jax.lax.broadcasted_iota(jnp.int32, logits.shape, 1)
    m1 = jnp.max(logits, axis=1, keepdims=True)
    i1 = jnp.min(jnp.where(logits == m1, iota_e, E), axis=1, keepdims=True)
    masked = jnp.where(iota_e == i1, -jnp.inf, logits)
    m2 = jnp.max(masked, axis=1, keepdims=True)
    i2 = jnp.min(jnp.where(masked == m2, iota_e, E), axis=1, keepdims=True)
    wa = jax.nn.sigmoid(m1 - m2)

    flat_e = jnp.concatenate([i1, i2], axis=0)
    flat_w = jnp.concatenate([wa, 1.0 - wa], axis=0)
    s_iota = jax.lax.broadcasted_iota(jnp.int32, (T * K, 1), 0)
    tid = jnp.where(s_iota >= T, s_iota - T, s_iota).astype(jnp.float32)

    lane_e = jax.lax.broadcasted_iota(jnp.int32, (T * K, E), 1)
    H = (flat_e == lane_e).astype(jnp.float32)

    r_i = jax.lax.broadcasted_iota(jnp.int32, (CH, CH), 0)
    c_i = jax.lax.broadcasted_iota(jnp.int32, (CH, CH), 1)
    L = (r_i >= c_i).astype(jnp.float32)
    tot = jnp.zeros((1, E), jnp.float32)
    chunks = []
    for c in range(NCH):
        cc = jnp.round(jax.lax.dot_general(
            L, H[c * CH:(c + 1) * CH, :], (((1,), (0,)), ((), ())),
            preferred_element_type=jnp.float32)) + tot
        chunks.append(cc)
        tot = cc[CH - 1:CH, :]
    cum = jnp.concatenate(chunks, axis=0)

    counts = tot
    padded = jnp.ceil(counts * (1.0 / BS)) * BS
    e_r = jax.lax.broadcasted_iota(jnp.int32, (E, E), 0)
    e_c = jax.lax.broadcasted_iota(jnp.int32, (E, E), 1)
    U = (e_r < e_c).astype(jnp.float32)
    offs = jnp.round(jax.lax.dot_general(
        padded, U, (((1,), (0,)), ((), ())),
        preferred_element_type=jnp.float32))

    my_rank = jnp.sum(cum * H, axis=1, keepdims=True) - 1.0
    offs_g = jnp.sum(offs * H, axis=1, keepdims=True)
    pos = jnp.round(offs_g + my_rank).astype(jnp.int32)

    tw = jnp.concatenate([tid, flat_w], axis=1)
    lane_b = jax.lax.broadcasted_iota(jnp.int32, (T * K, BS), 1)
    for blk in range(NB):
        Mb = (pos - blk * BS == lane_b).astype(jnp.float32)
        tw_blk = jax.lax.dot_general(
            tw, Mb, (((0,), (0,)), ((), ())),
            preferred_element_type=jnp.float32)
        ids_ref[pl.ds(blk, 1), :] = jnp.round(tw_blk[0:1, :]).astype(jnp.int32)
        sw_ref[pl.ds(blk, 1), :] = tw_blk[1:2, :]

    bstart = (jax.lax.broadcasted_iota(jnp.int32, (NB, 1), 0) * BS
              ).astype(jnp.float32)
    be_ref[...] = (jnp.sum(
        (bstart >= offs).astype(jnp.int32), axis=1, keepdims=True) - 1)
    total = (offs + padded)[:, E - 1:E]
    valid_ref[...] = (bstart < total).astype(jnp.int32)


def _moe_body(be_ref, ids_ref, valid_ref, w1_ref, w3_ref, w2_ref, x_ref,
              sw_ref, out_ref, xb_ref, acc_ref):
    b = pl.program_id(0)
    f = pl.program_id(1)

    @pl.when(jnp.logical_and(b == 0, f == 0))
    def _():
        out_ref[...] = jnp.zeros_like(out_ref)

    is_valid = valid_ref[b] > 0

    @pl.when(jnp.logical_and(f == 0, is_valid))
    def _():
        def gather(i, _):
            t = ids_ref[b * BS + i]
            xb_ref[pl.ds(i, 1), :] = x_ref[pl.ds(t, 1), :]
            return 0
        jax.lax.fori_loop(0, BS, gather, 0)

    @pl.when(is_valid)
    def _():
        xb = xb_ref[...]
        h1 = jax.lax.dot_general(
            xb, w1_ref[0], (((1,), (1,)), ((), ())),
            preferred_element_type=jnp.float32)
        h3 = jax.lax.dot_general(
            xb, w3_ref[0], (((1,), (1,)), ((), ())),
            preferred_element_type=jnp.float32)
        h = h1 * jax.nn.sigmoid(h1) * h3
        y = jax.lax.dot_general(
            h, w2_ref[0], (((1,), (1,)), ((), ())),
            preferred_element_type=jnp.float32)

        @pl.when(f == 0)
        def _():
            acc_ref[...] = y

        @pl.when(f > 0)
        def _():
            acc_ref[...] += y

        @pl.when(f == NF - 1)
        def _():
            acc_ref[...] = acc_ref[...] * sw_ref[...]

            def scatter(i, _):
                t = ids_ref[b * BS + i]
                out_ref[pl.ds(t, 1), :] += acc_ref[pl.ds(i, 1), :]
                return 0
            jax.lax.fori_loop(0, BS, scatter, 0)


@jax.jit
def kernel(hidden_states, gate_w, w1, w3, w2):
    orig_shape = hidden_states.shape
    x = hidden_states.reshape(T, D)

    ids2d, sw2d, be2d, valid2d = pl.pallas_call(
        _gate_body,
        out_shape=(
            jax.ShapeDtypeStruct((NB, BS), jnp.int32),
            jax.ShapeDtypeStruct((NB, BS), jnp.float32),
            jax.ShapeDtypeStruct((NB, 1), jnp.int32),
            jax.ShapeDtypeStruct((NB, 1), jnp.int32),
        ),
    )(x, gate_w)

    sort_ids = ids2d.reshape(PADDED)
    slot_w = sw2d.reshape(PADDED, 1)
    block_expert = be2d.reshape(NB)
    block_valid = valid2d.reshape(NB)

    grid_spec = pltpu.PrefetchScalarGridSpec(
        num_scalar_prefetch=3,
        grid=(NB, NF),
        in_specs=[
            pl.BlockSpec((1, FB, D), lambda b, f, be, ids, va: (be[b], f, 0)),
            pl.BlockSpec((1, FB, D), lambda b, f, be, ids, va: (be[b], f, 0)),
            pl.BlockSpec((1, D, FB), lambda b, f, be, ids, va: (be[b], 0, f)),
            pl.BlockSpec((T, D), lambda b, f, be, ids, va: (0, 0)),
            pl.BlockSpec((BS, 1), lambda b, f, be, ids, va: (b, 0)),
        ],
        out_specs=pl.BlockSpec((T, D), lambda b, f, be, ids, va: (0, 0)),
        scratch_shapes=[
            pltpu.VMEM((BS, D), jnp.float32),
            pltpu.VMEM((BS, D), jnp.float32),
        ],
    )
    y = pl.pallas_call(
        _moe_body,
        grid_spec=grid_spec,
        out_shape=jax.ShapeDtypeStruct((T, D), jnp.float32),
        compiler_params=pltpu.CompilerParams(
            dimension_semantics=("arbitrary", "arbitrary"),
        ),
    )(block_expert, sort_ids, block_valid, w1, w3, w2, x, slot_w)
    return y.reshape(orig_shape)
